# baseline (speedup 1.0000x reference)
import functools

import jax
import jax.numpy as jnp
from jax.experimental import pallas as pl
from jax.experimental.pallas import tpu as pltpu

_EPS = 1e-8
_LANE = 128


def _gln_kernel(x_ref, w_ref, b_ref, o_ref, scale_ref, shift_ref,
                *, eps, inv_n, tf):
    ph = pl.program_id(1)

    @pl.when(ph == 0)
    def _stats():
        x = x_ref[...]
        s = jnp.sum(x)
        q = jnp.sum(x * x)
        mean = s * inv_n
        var = jnp.maximum(q * inv_n - mean * mean, 0.0)
        inv_std = jax.lax.rsqrt(var + jnp.float32(eps))
        scale = w_ref[...] * inv_std
        scale_ref[...] = scale
        shift_ref[...] = b_ref[...] - mean * scale

    @pl.when(ph > 0)
    def _norm():
        start = (ph - 1) * tf
        xc = x_ref[:, pl.ds(start, tf)]
        o_ref[...] = xc * scale_ref[...] + shift_ref[...]


def kernel(x, weight, bias):
    orig_shape = x.shape
    if x.ndim == 4:
        N, C, K, S = x.shape
        F = K * S
    else:
        N, C, F = x.shape
    x3 = x.reshape(N, C, F)
    w = weight.reshape(C, 1).astype(jnp.float32)
    b = bias.reshape(C, 1).astype(jnp.float32)

    nchunks = 1
    for k in (5, 4, 2):
        if F % (k * _LANE) == 0:
            nchunks = k
            break
    tf = F // nchunks

    out = pl.pallas_call(
        functools.partial(_gln_kernel, eps=_EPS, inv_n=1.0 / (C * F), tf=tf),
        out_shape=jax.ShapeDtypeStruct((N, C, F), x.dtype),
        grid=(N, nchunks + 1),
        in_specs=[
            pl.BlockSpec((None, C, F), lambda n, ph: (n, 0, 0)),
            pl.BlockSpec((C, 1), lambda n, ph: (0, 0)),
            pl.BlockSpec((C, 1), lambda n, ph: (0, 0)),
        ],
        out_specs=pl.BlockSpec(
            (None, C, tf),
            lambda n, ph: (n, 0, jnp.maximum(ph - 1, 0))),
        scratch_shapes=[
            pltpu.VMEM((C, 1), jnp.float32),
            pltpu.VMEM((C, 1), jnp.float32),
        ],
        compiler_params=pltpu.CompilerParams(
            dimension_semantics=("parallel", "arbitrary"),
            vmem_limit_bytes=48 * 1024 * 1024),
    )(x3, w, b)
    return out.reshape(orig_shape)


# speedup vs baseline: 1.2372x; 1.2372x over previous
import functools

import jax
import jax.numpy as jnp
from jax.experimental import pallas as pl
from jax.experimental.pallas import tpu as pltpu

_EPS = 1e-8


def _gln_kernel(x_ref, w_ref, b_ref, o_ref, scale_ref, shift_ref,
                *, eps, inv_n):
    ph = pl.program_id(1)

    @pl.when(ph == 0)
    def _stats():
        x = x_ref[...]
        s = jnp.sum(x)
        q = jnp.sum(x * x)
        mean = s * inv_n
        var = jnp.maximum(q * inv_n - mean * mean, 0.0)
        inv_std = jax.lax.rsqrt(var + jnp.float32(eps))
        scale = w_ref[...] * inv_std
        scale_ref[...] = scale
        shift_ref[...] = b_ref[...] - mean * scale

    @pl.when(ph > 0)
    def _norm():
        o_ref[...] = x_ref[...] * scale_ref[...] + shift_ref[...]


def kernel(x, weight, bias):
    orig_shape = x.shape
    if x.ndim == 4:
        N, C, K, S = x.shape
        F = K * S
    else:
        N, C, F = x.shape
    x3 = x.reshape(N, C, F)
    w = weight.reshape(C, 1).astype(jnp.float32)
    b = bias.reshape(C, 1).astype(jnp.float32)

    out = pl.pallas_call(
        functools.partial(_gln_kernel, eps=_EPS, inv_n=1.0 / (C * F)),
        out_shape=jax.ShapeDtypeStruct((N, C, F), x.dtype),
        grid=(N, 2),
        in_specs=[
            pl.BlockSpec((None, C, F), lambda n, ph: (n, 0, 0)),
            pl.BlockSpec((C, 1), lambda n, ph: (0, 0)),
            pl.BlockSpec((C, 1), lambda n, ph: (0, 0)),
        ],
        out_specs=pl.BlockSpec((None, C, F), lambda n, ph: (n, 0, 0)),
        scratch_shapes=[
            pltpu.VMEM((C, 1), jnp.float32),
            pltpu.VMEM((C, 1), jnp.float32),
        ],
        compiler_params=pltpu.CompilerParams(
            dimension_semantics=("parallel", "arbitrary"),
            vmem_limit_bytes=48 * 1024 * 1024),
    )(x3, w, b)
    return out.reshape(orig_shape)
